# baseline (speedup 1.0000x reference)
import jax
import jax.numpy as jnp
import numpy as np
from jax.experimental import pallas as pl
from jax.sharding import Mesh, NamedSharding, PartitionSpec as P

N = 10000
D = 64
BM = 200


def _matmul_block(a_ref, e_ref, o_ref):
    a16 = a_ref[...].astype(jnp.bfloat16)
    e16 = e_ref[...].astype(jnp.bfloat16)
    o_ref[...] = jnp.dot(a16, e16, preferred_element_type=jnp.float32)


def _local_matmul(a_local, e_full):
    m_local = a_local.shape[0]
    return pl.pallas_call(
        _matmul_block,
        grid=(m_local // BM,),
        in_specs=[
            pl.BlockSpec((BM, N), lambda i: (i, 0)),
            pl.BlockSpec((N, D), lambda i: (0, 0)),
        ],
        out_specs=pl.BlockSpec((BM, D), lambda i: (i, 0)),
        out_shape=jax.ShapeDtypeStruct((m_local, D), jnp.float32),
    )(a_local, e_full)


def kernel(A_hat, E):
    devs = jax.devices()
    n_shards = 2 if len(devs) >= 2 and N % (2 * BM) == 0 else 1
    if n_shards == 1:
        return _local_matmul(A_hat, E)
    mesh = Mesh(np.array(devs[:n_shards]), ("x",))
    A_s = jax.lax.with_sharding_constraint(
        A_hat, NamedSharding(mesh, P("x", None)))
    E_s = jax.lax.with_sharding_constraint(E, NamedSharding(mesh, P()))
    out = jax.shard_map(
        _local_matmul, mesh=mesh,
        in_specs=(P("x", None), P()),
        out_specs=P("x", None),
        check_vma=False,
    )(A_s, E_s)
    return out


# speedup vs baseline: 4.9270x; 4.9270x over previous
import jax
import jax.numpy as jnp
from jax.experimental import pallas as pl

N = 10000
D = 64
BM = 400


def _matmul_block(a_ref, e_ref, o_ref):
    a16 = a_ref[...].astype(jnp.bfloat16)
    e16 = e_ref[...].astype(jnp.bfloat16)
    o_ref[...] = jnp.dot(a16, e16, preferred_element_type=jnp.float32)


def kernel(A_hat, E):
    return pl.pallas_call(
        _matmul_block,
        grid=(N // BM,),
        in_specs=[
            pl.BlockSpec((BM, N), lambda i: (i, 0)),
            pl.BlockSpec((N, D), lambda i: (0, 0)),
        ],
        out_specs=pl.BlockSpec((BM, D), lambda i: (i, 0)),
        out_shape=jax.ShapeDtypeStruct((N, D), jnp.float32),
    )(A_hat, E)


# speedup vs baseline: 4.9371x; 1.0020x over previous
import jax
import jax.numpy as jnp
from jax.experimental import pallas as pl
from jax.experimental.pallas import tpu as pltpu

N = 10000
D = 64
BM = 400


def _matmul_block(a_ref, e_ref, o_ref):
    a16 = a_ref[...].astype(jnp.bfloat16)
    e16 = e_ref[...].astype(jnp.bfloat16)
    o_ref[...] = jnp.dot(a16, e16, preferred_element_type=jnp.float32)


def kernel(A_hat, E):
    return pl.pallas_call(
        _matmul_block,
        grid=(N // BM,),
        in_specs=[
            pl.BlockSpec((BM, N), lambda i: (i, 0)),
            pl.BlockSpec((N, D), lambda i: (0, 0)),
        ],
        out_specs=pl.BlockSpec((BM, D), lambda i: (i, 0)),
        out_shape=jax.ShapeDtypeStruct((N, D), jnp.float32),
        compiler_params=pltpu.CompilerParams(
            dimension_semantics=("parallel",),
        ),
    )(A_hat, E)


# speedup vs baseline: 4.9532x; 1.0033x over previous
import jax
import jax.numpy as jnp
from jax.experimental import pallas as pl
from jax.experimental.pallas import tpu as pltpu

N = 10000
D = 64
BM = 400


def _matmul_block(a_ref, e_ref, o_ref):
    o_ref[...] = jnp.dot(
        a_ref[...],
        e_ref[...],
        precision=jax.lax.Precision.DEFAULT,
        preferred_element_type=jnp.float32,
    )


def kernel(A_hat, E):
    return pl.pallas_call(
        _matmul_block,
        grid=(N // BM,),
        in_specs=[
            pl.BlockSpec((BM, N), lambda i: (i, 0)),
            pl.BlockSpec((N, D), lambda i: (0, 0)),
        ],
        out_specs=pl.BlockSpec((BM, D), lambda i: (i, 0)),
        out_shape=jax.ShapeDtypeStruct((N, D), jnp.float32),
        compiler_params=pltpu.CompilerParams(
            dimension_semantics=("parallel",),
        ),
    )(A_hat, E)


# speedup vs baseline: 4.9779x; 1.0050x over previous
import jax
import jax.numpy as jnp
from jax.experimental import pallas as pl
from jax.experimental.pallas import tpu as pltpu

N = 10000
D = 64
BM = 200


def _matmul_block(a_ref, e_ref, o_ref):
    o_ref[...] = jnp.dot(
        a_ref[...],
        e_ref[...],
        precision=jax.lax.Precision.DEFAULT,
        preferred_element_type=jnp.float32,
    )


def kernel(A_hat, E):
    return pl.pallas_call(
        _matmul_block,
        grid=(N // BM,),
        in_specs=[
            pl.BlockSpec((BM, N), lambda i: (i, 0)),
            pl.BlockSpec((N, D), lambda i: (0, 0)),
        ],
        out_specs=pl.BlockSpec((BM, D), lambda i: (i, 0)),
        out_shape=jax.ShapeDtypeStruct((N, D), jnp.float32),
        compiler_params=pltpu.CompilerParams(
            dimension_semantics=("parallel",),
        ),
    )(A_hat, E)
